# baseline (speedup 1.0000x reference)
import functools

import jax
import jax.numpy as jnp
from jax import lax
from jax.experimental import pallas as pl
from jax.experimental.pallas import tpu as pltpu
from jax.experimental.pallas import tpu_sc as plsc

BATCH = 16384
EMBED_DIM = 32
SDAE_DIM = 16
OUT_DIM = EMBED_DIM + SDAE_DIM
TROWS = 1000000

_INFO = plsc.get_sparse_core_info()
_NC = _INFO.num_cores
_NS = _INFO.num_subcores
_NW = _NC * _NS
_BPW = BATCH // _NW
_NSG = _BPW // 16

_mesh = plsc.VectorSubcoreMesh(core_axis_name="c", subcore_axis_name="s")


@functools.partial(
    pl.kernel,
    mesh=_mesh,
    out_type=jax.ShapeDtypeStruct((OUT_DIM, BATCH), jnp.float32),
    compiler_params=pltpu.CompilerParams(needs_layout_passes=False),
    scratch_types=[
        pltpu.VMEM((_BPW,), jnp.int32),
        pltpu.VMEM((_BPW,), jnp.int32),
        pltpu.VMEM((8, EMBED_DIM, 128), jnp.float32),
        pltpu.VMEM((8, EMBED_DIM, 128), jnp.float32),
        pltpu.VMEM((SDAE_DIM, _BPW), jnp.float32),
        pltpu.VMEM((SDAE_DIM, _BPW), jnp.float32),
        pltpu.VMEM((OUT_DIM, _BPW), jnp.float32),
        pltpu.SemaphoreType.DMA,
        pltpu.SemaphoreType.DMA,
    ],
)
def _gmf_sc(uidx_hbm, iidx_hbm, usdt_hbm, isdt_hbm, utt_hbm, itt_hbm,
            outt_hbm, uidx_v, iidx_v, uslab_v, islab_v, usd_v, isd_v,
            out_v, gsem0, gsem1):
    wid = lax.axis_index("s") * _NC + lax.axis_index("c")
    base = wid * _BPW

    pltpu.sync_copy(uidx_hbm.at[pl.ds(base, _BPW)], uidx_v)
    pltpu.sync_copy(iidx_hbm.at[pl.ds(base, _BPW)], iidx_v)
    pltpu.sync_copy(usdt_hbm.at[:, pl.ds(base, _BPW)], usd_v)
    pltpu.sync_copy(isdt_hbm.at[:, pl.ds(base, _BPW)], isd_v)

    d_lo = lax.iota(jnp.int32, 16)
    d_hi = d_lo + 16
    sems = (gsem0, gsem1)

    def sg_body(sg, carry):
        u16 = uidx_v[pl.ds(sg * 16, 16)]
        i16 = iidx_v[pl.ds(sg * 16, 16)]
        su16 = u16 & ~jnp.int32(127)
        si16 = i16 & ~jnp.int32(127)
        cu16 = u16 & jnp.int32(127)
        ci16 = i16 & jnp.int32(127)

        def fire(q):
            p = q & 1
            copies = []
            for s in range(4):
                k = q * 4 + s
                slot = p * 4 + s
                su = pl.multiple_of(su16[k], 128)
                si = pl.multiple_of(si16[k], 128)
                copies.append(pltpu.async_copy(
                    utt_hbm.at[:, pl.ds(su, 128)], uslab_v.at[slot], sems[p]))
                copies.append(pltpu.async_copy(
                    itt_hbm.at[:, pl.ds(si, 128)], islab_v.at[slot], sems[p]))
            return copies

        def extract(q):
            p = q & 1
            for s in range(4):
                k = q * 4 + s
                slot = jnp.full((16,), p * 4 + s, jnp.int32)
                cu = jnp.full((16,), cu16[k], jnp.int32)
                ci = jnp.full((16,), ci16[k], jnp.int32)
                col = jnp.full((16,), sg * 16 + k, jnp.int32)
                u_lo = plsc.load_gather(uslab_v, [slot, d_lo, cu])
                u_hi = plsc.load_gather(uslab_v, [slot, d_hi, cu])
                i_lo = plsc.load_gather(islab_v, [slot, d_lo, ci])
                i_hi = plsc.load_gather(islab_v, [slot, d_hi, ci])
                plsc.store_scatter(out_v, [d_lo, col], u_lo * i_lo)
                plsc.store_scatter(out_v, [d_hi, col], u_hi * i_hi)

        c0 = fire(0)
        c1 = fire(1)
        for d in range(SDAE_DIM):
            out_v[EMBED_DIM + d, pl.ds(sg * 16, 16)] = (
                usd_v[d, pl.ds(sg * 16, 16)] * isd_v[d, pl.ds(sg * 16, 16)])
        for cpy in c0:
            cpy.wait()
        extract(0)
        c2 = fire(2)
        for cpy in c1:
            cpy.wait()
        extract(1)
        c3 = fire(3)
        for cpy in c2:
            cpy.wait()
        extract(2)
        for cpy in c3:
            cpy.wait()
        extract(3)
        return carry

    lax.fori_loop(0, _NSG, sg_body, None)

    pltpu.sync_copy(out_v, outt_hbm.at[:, pl.ds(base, _BPW)])


def kernel(user_indices, item_indices, user_sdae_feat, item_sdae_feat,
           user_table, item_table):
    uidx = user_indices.astype(jnp.int32)
    iidx = item_indices.astype(jnp.int32)
    out_t = _gmf_sc(uidx, iidx, user_sdae_feat.T, item_sdae_feat.T,
                    user_table.T, item_table.T)
    return out_t.T


# speedup vs baseline: 1.0033x; 1.0033x over previous
import functools

import jax
import jax.numpy as jnp
from jax import lax
from jax.experimental import pallas as pl
from jax.experimental.pallas import tpu as pltpu
from jax.experimental.pallas import tpu_sc as plsc

BATCH = 16384
EMBED_DIM = 32
SDAE_DIM = 16
OUT_DIM = EMBED_DIM + SDAE_DIM

_INFO = plsc.get_sparse_core_info()
_NC = _INFO.num_cores
_NS = _INFO.num_subcores
_NW = _NC * _NS
_BPW = BATCH // _NW
_NSG = _BPW // 16

_mesh = plsc.VectorSubcoreMesh(core_axis_name="c", subcore_axis_name="s")


@functools.partial(
    pl.kernel,
    mesh=_mesh,
    out_type=jax.ShapeDtypeStruct((OUT_DIM, BATCH), jnp.float32),
    compiler_params=pltpu.CompilerParams(needs_layout_passes=False),
    scratch_types=[
        pltpu.VMEM((_BPW,), jnp.int32),
        pltpu.VMEM((_BPW,), jnp.int32),
        pltpu.VMEM((8, EMBED_DIM, 128), jnp.float32),
        pltpu.VMEM((8, EMBED_DIM, 128), jnp.float32),
        pltpu.VMEM((SDAE_DIM, _BPW), jnp.float32),
        pltpu.VMEM((SDAE_DIM, _BPW), jnp.float32),
        pltpu.VMEM((OUT_DIM, _BPW), jnp.float32),
        pltpu.SemaphoreType.DMA,
        pltpu.SemaphoreType.DMA,
    ],
)
def _gmf_sc(uidx_hbm, iidx_hbm, usdt_hbm, isdt_hbm, utt_hbm, itt_hbm,
            outt_hbm, uidx_v, iidx_v, uslab_v, islab_v, usd_v, isd_v,
            out_v, gsem0, gsem1):
    wid = lax.axis_index("s") * _NC + lax.axis_index("c")
    base = wid * _BPW

    pltpu.sync_copy(uidx_hbm.at[pl.ds(base, _BPW)], uidx_v)
    pltpu.sync_copy(iidx_hbm.at[pl.ds(base, _BPW)], iidx_v)
    pltpu.sync_copy(usdt_hbm.at[:, pl.ds(base, _BPW)], usd_v)
    pltpu.sync_copy(isdt_hbm.at[:, pl.ds(base, _BPW)], isd_v)

    d_lo = lax.iota(jnp.int32, 16)
    d_hi = d_lo + 16
    sems = (gsem0, gsem1)

    def sg_body(sg, carry):
        u16 = uidx_v[pl.ds(sg * 16, 16)]
        i16 = iidx_v[pl.ds(sg * 16, 16)]
        su16 = u16 & ~jnp.int32(127)
        si16 = i16 & ~jnp.int32(127)
        cu16 = u16 & jnp.int32(127)
        ci16 = i16 & jnp.int32(127)

        def fire(q):
            p = q & 1
            copies = []
            for s in range(4):
                k = q * 4 + s
                slot = p * 4 + s
                su = pl.multiple_of(su16[k], 128)
                si = pl.multiple_of(si16[k], 128)
                copies.append(pltpu.async_copy(
                    utt_hbm.at[:, pl.ds(su, 128)], uslab_v.at[slot], sems[p]))
                copies.append(pltpu.async_copy(
                    itt_hbm.at[:, pl.ds(si, 128)], islab_v.at[slot], sems[p]))
            return copies

        def extract(q):
            p = q & 1
            for s in range(4):
                k = q * 4 + s
                slot = jnp.full((16,), p * 4 + s, jnp.int32)
                cu = jnp.full((16,), cu16[k], jnp.int32)
                ci = jnp.full((16,), ci16[k], jnp.int32)
                col = jnp.full((16,), sg * 16 + k, jnp.int32)
                u_lo = plsc.load_gather(uslab_v, [slot, d_lo, cu])
                u_hi = plsc.load_gather(uslab_v, [slot, d_hi, cu])
                i_lo = plsc.load_gather(islab_v, [slot, d_lo, ci])
                i_hi = plsc.load_gather(islab_v, [slot, d_hi, ci])
                plsc.store_scatter(out_v, [d_lo, col], u_lo * i_lo)
                plsc.store_scatter(out_v, [d_hi, col], u_hi * i_hi)

        c0 = fire(0)
        c1 = fire(1)
        for d in range(SDAE_DIM):
            out_v[EMBED_DIM + d, pl.ds(sg * 16, 16)] = (
                usd_v[d, pl.ds(sg * 16, 16)] * isd_v[d, pl.ds(sg * 16, 16)])
        for cpy in c0:
            cpy.wait()
        extract(0)
        c2 = fire(2)
        for cpy in c1:
            cpy.wait()
        extract(1)
        c3 = fire(3)
        for cpy in c2:
            cpy.wait()
        extract(2)
        for cpy in c3:
            cpy.wait()
        extract(3)
        return carry

    lax.fori_loop(0, _NSG, sg_body, None)

    pltpu.sync_copy(out_v, outt_hbm.at[:, pl.ds(base, _BPW)])


def kernel(user_indices, item_indices, user_sdae_feat, item_sdae_feat,
           user_table, item_table):
    uidx = user_indices.astype(jnp.int32)
    iidx = item_indices.astype(jnp.int32)
    out_t = _gmf_sc(uidx, iidx, user_sdae_feat.T, item_sdae_feat.T,
                    user_table.T, item_table.T)
    return out_t.T
